# baseline (speedup 1.0000x reference)
import jax
import jax.numpy as jnp
from jax.experimental import pallas as pl

_B, _N, _C = 8, 2048, 96
_HID, _HOPS, _NH, _K = 192, 3, 3, 8
_DH = _C // _NH
_RT = 1024
_LT = 512


def _hop_kernel(rows_ref, cur_ref, curT_ref, w1t_ref, b1_ref, w2t_ref, b2_ref,
                enc_ref, encT_ref):
    rows = rows_ref[0]
    cur = cur_ref[0]
    curT = curT_ref[0]
    sq_c = jnp.sum(curT * curT, axis=0, keepdims=True)
    cur_hi = cur.astype(jnp.bfloat16).astype(jnp.float32)
    cur_lo = cur - cur_hi
    dot = jnp.dot(rows, curT, preferred_element_type=jnp.float32)
    sq_r = jnp.sum(rows * rows, axis=1, keepdims=True)
    d = jnp.sqrt(jnp.maximum(sq_r + sq_c - 2.0 * dot, 0.0))
    iota = jax.lax.broadcasted_iota(jnp.int32, d.shape, 1)
    mask = jnp.zeros(d.shape, jnp.float32)
    for _ in range(_K):
        m = jnp.min(d, axis=1, keepdims=True)
        fidx = jnp.min(jnp.where(d == m, iota, _N), axis=1, keepdims=True)
        chosen = iota == fidx
        mask = jnp.where(chosen, 1.0, mask)
        d = jnp.where(chosen, jnp.inf, d)
    nb = (jnp.dot(mask, cur_hi, preferred_element_type=jnp.float32)
          + jnp.dot(mask, cur_lo, preferred_element_type=jnp.float32)) * (1.0 / _K)
    hdn = jnp.dot(nb, w1t_ref[...], preferred_element_type=jnp.float32) + b1_ref[...]
    hdn = jnp.where(hdn >= 0.0, hdn, 0.2 * hdn)
    enc = jnp.dot(hdn, w2t_ref[...], preferred_element_type=jnp.float32) + b2_ref[...]
    enc_ref[0] = enc
    encT_ref[0] = enc.T


def _run_hop(cur, curT, w1t, b1, w2t, b2):
    return pl.pallas_call(
        _hop_kernel,
        grid=(_B, _N // _RT),
        in_specs=[
            pl.BlockSpec((1, _RT, _C), lambda b, r: (b, r, 0)),
            pl.BlockSpec((1, _N, _C), lambda b, r: (b, 0, 0)),
            pl.BlockSpec((1, _C, _N), lambda b, r: (b, 0, 0)),
            pl.BlockSpec((_C, _HID), lambda b, r: (0, 0)),
            pl.BlockSpec((1, _HID), lambda b, r: (0, 0)),
            pl.BlockSpec((_HID, _C), lambda b, r: (0, 0)),
            pl.BlockSpec((1, _C), lambda b, r: (0, 0)),
        ],
        out_specs=[
            pl.BlockSpec((1, _RT, _C), lambda b, r: (b, r, 0)),
            pl.BlockSpec((1, _C, _RT), lambda b, r: (b, 0, r)),
        ],
        out_shape=[
            jax.ShapeDtypeStruct((_B, _N, _C), jnp.float32),
            jax.ShapeDtypeStruct((_B, _C, _N), jnp.float32),
        ],
    )(cur, cur, curT, w1t, b1, w2t, b2)


def _mha_kernel(f0_ref, f1_ref, f2_ref, f3_ref, wq_ref, wk_ref, wv_ref,
                bq_ref, bk_ref, bv_ref, owt_ref, ob_ref, out_ref):
    feats = (f0_ref[...], f1_ref[...], f2_ref[...], f3_ref[...])
    wq, wk, wv = wq_ref[...], wk_ref[...], wv_ref[...]
    q = [jnp.dot(f, wq, preferred_element_type=jnp.float32) + bq_ref[...]
         for f in feats]
    k = [jnp.dot(f, wk, preferred_element_type=jnp.float32) + bk_ref[...]
         for f in feats]
    v = [jnp.dot(f, wv, preferred_element_type=jnp.float32) + bv_ref[...]
         for f in feats]
    seg = (jax.lax.broadcasted_iota(jnp.int32, (_C, _NH), 0) // _DH
           == jax.lax.broadcasted_iota(jnp.int32, (_C, _NH), 1)).astype(jnp.float32)
    segT = (jax.lax.broadcasted_iota(jnp.int32, (_NH, _C), 0)
            == jax.lax.broadcasted_iota(jnp.int32, (_NH, _C), 1) // _DH
            ).astype(jnp.float32)
    scale = 1.0 / (_DH ** 0.5)
    T = _HOPS + 1
    logits = [[jnp.dot(q[t] * k[s], seg, preferred_element_type=jnp.float32) * scale
               for s in range(T)] for t in range(T)]
    wbar = [jnp.zeros_like(logits[0][0]) for _ in range(T)]
    for t in range(T):
        m = logits[t][0]
        for s in range(1, T):
            m = jnp.maximum(m, logits[t][s])
        e = [jnp.exp(logits[t][s] - m) for s in range(T)]
        tot = e[0]
        for s in range(1, T):
            tot = tot + e[s]
        inv = (1.0 / T) / tot
        for s in range(T):
            wbar[s] = wbar[s] + e[s] * inv
    o = jnp.zeros_like(v[0])
    for s in range(T):
        o = o + jnp.dot(wbar[s], segT, preferred_element_type=jnp.float32) * v[s]
    out_ref[...] = jnp.dot(o, owt_ref[...], preferred_element_type=jnp.float32) + ob_ref[...]


def _run_mha(fs, wq, wk, wv, bq, bk, bv, owt, ob):
    L = _B * _N
    fspec = pl.BlockSpec((_LT, _C), lambda l: (l, 0))
    wspec = pl.BlockSpec((_C, _C), lambda l: (0, 0))
    bspec = pl.BlockSpec((1, _C), lambda l: (0, 0))
    return pl.pallas_call(
        _mha_kernel,
        grid=(L // _LT,),
        in_specs=[fspec, fspec, fspec, fspec, wspec, wspec, wspec,
                  bspec, bspec, bspec, wspec, bspec],
        out_specs=pl.BlockSpec((_LT, _C), lambda l: (l, 0)),
        out_shape=jax.ShapeDtypeStruct((L, _C), jnp.float32),
    )(*fs, wq, wk, wv, bq, bk, bv, owt, ob)


def kernel(xyz, params):
    feats = [xyz]
    cur = xyz
    curT = jnp.transpose(xyz, (0, 2, 1))
    for h in range(_HOPS):
        W1, b1, W2, b2 = params["hops"][h]
        enc, encT = _run_hop(cur, curT, W1.T, b1.reshape(1, _HID),
                             W2.T, b2.reshape(1, _C))
        feats.append(enc)
        cur, curT = enc, encT
    L = _B * _N
    fs = [f.reshape(L, _C) for f in feats]
    in_w, in_b = params["in_w"], params["in_b"]
    wq = in_w[0:_C].T
    wk = in_w[_C:2 * _C].T
    wv = in_w[2 * _C:3 * _C].T
    bq = in_b[0:_C].reshape(1, _C)
    bk = in_b[_C:2 * _C].reshape(1, _C)
    bv = in_b[2 * _C:3 * _C].reshape(1, _C)
    owt = params["out_w"].T
    ob = params["out_b"].reshape(1, _C)
    out = _run_mha(fs, wq, wk, wv, bq, bk, bv, owt, ob)
    return out.reshape(_B, _N, _C)


# speedup vs baseline: 1.0599x; 1.0599x over previous
import jax
import jax.numpy as jnp
from jax.experimental import pallas as pl

_B, _N, _C = 8, 2048, 96
_HID, _HOPS, _NH, _K = 192, 3, 3, 8
_DH = _C // _NH
_RT = 512
_LT = 512


def _hop_kernel(rows_ref, cur_ref, curT_ref, w1t_ref, b1_ref, w2t_ref, b2_ref,
                enc_ref, encT_ref):
    rows = rows_ref[0]
    cur = cur_ref[0]
    curT = curT_ref[0]
    sq_c = jnp.sum(curT * curT, axis=0, keepdims=True)
    cur_hi = cur.astype(jnp.bfloat16).astype(jnp.float32)
    cur_lo = cur - cur_hi
    dot = jnp.dot(rows, curT, preferred_element_type=jnp.float32)
    sq_r = jnp.sum(rows * rows, axis=1, keepdims=True)
    d = jnp.maximum(sq_r + sq_c - 2.0 * dot, 0.0)
    iota = jax.lax.broadcasted_iota(jnp.int32, d.shape, 1)
    mask = jnp.zeros(d.shape, jnp.float32)
    for _ in range(_K):
        m = jnp.min(d, axis=1, keepdims=True)
        fidx = jnp.min(jnp.where(d == m, iota, _N), axis=1, keepdims=True)
        chosen = iota == fidx
        mask = jnp.where(chosen, 1.0, mask)
        d = jnp.where(chosen, jnp.inf, d)
    nb = (jnp.dot(mask, cur_hi, preferred_element_type=jnp.float32)
          + jnp.dot(mask, cur_lo, preferred_element_type=jnp.float32)) * (1.0 / _K)
    hdn = jnp.dot(nb, w1t_ref[...], preferred_element_type=jnp.float32) + b1_ref[...]
    hdn = jnp.where(hdn >= 0.0, hdn, 0.2 * hdn)
    enc = jnp.dot(hdn, w2t_ref[...], preferred_element_type=jnp.float32) + b2_ref[...]
    enc_ref[0] = enc
    encT_ref[0] = enc.T


def _run_hop(cur, curT, w1t, b1, w2t, b2):
    return pl.pallas_call(
        _hop_kernel,
        grid=(_B, _N // _RT),
        in_specs=[
            pl.BlockSpec((1, _RT, _C), lambda b, r: (b, r, 0)),
            pl.BlockSpec((1, _N, _C), lambda b, r: (b, 0, 0)),
            pl.BlockSpec((1, _C, _N), lambda b, r: (b, 0, 0)),
            pl.BlockSpec((_C, _HID), lambda b, r: (0, 0)),
            pl.BlockSpec((1, _HID), lambda b, r: (0, 0)),
            pl.BlockSpec((_HID, _C), lambda b, r: (0, 0)),
            pl.BlockSpec((1, _C), lambda b, r: (0, 0)),
        ],
        out_specs=[
            pl.BlockSpec((1, _RT, _C), lambda b, r: (b, r, 0)),
            pl.BlockSpec((1, _C, _RT), lambda b, r: (b, 0, r)),
        ],
        out_shape=[
            jax.ShapeDtypeStruct((_B, _N, _C), jnp.float32),
            jax.ShapeDtypeStruct((_B, _C, _N), jnp.float32),
        ],
    )(cur, cur, curT, w1t, b1, w2t, b2)


def _mha_kernel(f0_ref, f1_ref, f2_ref, f3_ref, wq_ref, wk_ref, wv_ref,
                bq_ref, bk_ref, bv_ref, owt_ref, ob_ref, out_ref):
    feats = (f0_ref[...], f1_ref[...], f2_ref[...], f3_ref[...])
    wq, wk, wv = wq_ref[...], wk_ref[...], wv_ref[...]
    q = [jnp.dot(f, wq, preferred_element_type=jnp.float32) + bq_ref[...]
         for f in feats]
    k = [jnp.dot(f, wk, preferred_element_type=jnp.float32) + bk_ref[...]
         for f in feats]
    v = [jnp.dot(f, wv, preferred_element_type=jnp.float32) + bv_ref[...]
         for f in feats]
    seg = (jax.lax.broadcasted_iota(jnp.int32, (_C, _NH), 0) // _DH
           == jax.lax.broadcasted_iota(jnp.int32, (_C, _NH), 1)).astype(jnp.float32)
    segT = (jax.lax.broadcasted_iota(jnp.int32, (_NH, _C), 0)
            == jax.lax.broadcasted_iota(jnp.int32, (_NH, _C), 1) // _DH
            ).astype(jnp.float32)
    scale = 1.0 / (_DH ** 0.5)
    T = _HOPS + 1
    logits = [[jnp.dot(q[t] * k[s], seg, preferred_element_type=jnp.float32) * scale
               for s in range(T)] for t in range(T)]
    wbar = [jnp.zeros_like(logits[0][0]) for _ in range(T)]
    for t in range(T):
        m = logits[t][0]
        for s in range(1, T):
            m = jnp.maximum(m, logits[t][s])
        e = [jnp.exp(logits[t][s] - m) for s in range(T)]
        tot = e[0]
        for s in range(1, T):
            tot = tot + e[s]
        inv = (1.0 / T) / tot
        for s in range(T):
            wbar[s] = wbar[s] + e[s] * inv
    o = jnp.zeros_like(v[0])
    for s in range(T):
        o = o + jnp.dot(wbar[s], segT, preferred_element_type=jnp.float32) * v[s]
    out_ref[...] = jnp.dot(o, owt_ref[...], preferred_element_type=jnp.float32) + ob_ref[...]


def _run_mha(fs, wq, wk, wv, bq, bk, bv, owt, ob):
    L = _B * _N
    fspec = pl.BlockSpec((_LT, _C), lambda l: (l, 0))
    wspec = pl.BlockSpec((_C, _C), lambda l: (0, 0))
    bspec = pl.BlockSpec((1, _C), lambda l: (0, 0))
    return pl.pallas_call(
        _mha_kernel,
        grid=(L // _LT,),
        in_specs=[fspec, fspec, fspec, fspec, wspec, wspec, wspec,
                  bspec, bspec, bspec, wspec, bspec],
        out_specs=pl.BlockSpec((_LT, _C), lambda l: (l, 0)),
        out_shape=jax.ShapeDtypeStruct((L, _C), jnp.float32),
    )(*fs, wq, wk, wv, bq, bk, bv, owt, ob)


def kernel(xyz, params):
    feats = [xyz]
    cur = xyz
    curT = jnp.transpose(xyz, (0, 2, 1))
    for h in range(_HOPS):
        W1, b1, W2, b2 = params["hops"][h]
        enc, encT = _run_hop(cur, curT, W1.T, b1.reshape(1, _HID),
                             W2.T, b2.reshape(1, _C))
        feats.append(enc)
        cur, curT = enc, encT
    L = _B * _N
    fs = [f.reshape(L, _C) for f in feats]
    in_w, in_b = params["in_w"], params["in_b"]
    wq = in_w[0:_C].T
    wk = in_w[_C:2 * _C].T
    wv = in_w[2 * _C:3 * _C].T
    bq = in_b[0:_C].reshape(1, _C)
    bk = in_b[_C:2 * _C].reshape(1, _C)
    bv = in_b[2 * _C:3 * _C].reshape(1, _C)
    owt = params["out_w"].T
    ob = params["out_b"].reshape(1, _C)
    out = _run_mha(fs, wq, wk, wv, bq, bk, bv, owt, ob)
    return out.reshape(_B, _N, _C)
